# baseline (speedup 1.0000x reference)
import jax
import jax.numpy as jnp
from jax.experimental import pallas as pl
from jax.experimental.pallas import tpu as pltpu


def _make_kernel(B, N, d_in, d_hid, d_out, TR):
    I = N // TR

    def _gcn_kernel(x_ref, adj_ref, w1_ref, b1_ref, w2_ref, b2_ref,
                    out_ref, t_s, g_s, cs_s):
        i = pl.program_id(1)
        rows = adj_ref[...]

        @pl.when(i == 0)
        def _init():
            t_s[...] = jnp.dot(x_ref[...], w1_ref[...],
                               preferred_element_type=jnp.float32)
            cs_s[...] = jnp.zeros_like(cs_s)

        h = jnp.dot(rows, t_s[...],
                    preferred_element_type=jnp.float32) + b1_ref[...]
        g_s[pl.ds(i * TR, TR), :] = jnp.where(h >= 0.0, h, 0.01 * h)
        cs_s[...] += jnp.sum(rows, axis=0, keepdims=True)

        @pl.when(i == I - 1)
        def _fini():
            r = cs_s[...] * (1.0 / N)
            v = jnp.dot(r, g_s[...], preferred_element_type=jnp.float32)
            out_ref[0] = (jnp.dot(v, w2_ref[...],
                                  preferred_element_type=jnp.float32)
                          + b2_ref[...])

    return _gcn_kernel


def kernel(x, graph_batch, W1, b1, W2, b2):
    B, N, d_in = x.shape
    d_hid = W1.shape[1]
    d_out = W2.shape[1]
    TR = 256
    I = N // TR
    x2 = x.reshape(B * N, d_in)
    adj2 = graph_batch.reshape(B * N, N)
    b1r = b1.reshape(1, d_hid)
    b2r = b2.reshape(1, d_out)
    return pl.pallas_call(
        _make_kernel(B, N, d_in, d_hid, d_out, TR),
        grid=(B, I),
        in_specs=[
            pl.BlockSpec((N, d_in), lambda b, i: (b, 0)),
            pl.BlockSpec((TR, N), lambda b, i: (b * I + i, 0)),
            pl.BlockSpec((d_in, d_hid), lambda b, i: (0, 0)),
            pl.BlockSpec((1, d_hid), lambda b, i: (0, 0)),
            pl.BlockSpec((d_hid, d_out), lambda b, i: (0, 0)),
            pl.BlockSpec((1, d_out), lambda b, i: (0, 0)),
        ],
        out_specs=pl.BlockSpec((1, 1, d_out), lambda b, i: (b, 0, 0)),
        out_shape=jax.ShapeDtypeStruct((B, 1, d_out), jnp.float32),
        scratch_shapes=[
            pltpu.VMEM((N, d_hid), jnp.float32),
            pltpu.VMEM((N, d_hid), jnp.float32),
            pltpu.VMEM((1, N), jnp.float32),
        ],
    )(x2, adj2, W1, b1r, W2, b2r).reshape(B, d_out)


# speedup vs baseline: 1.8862x; 1.8862x over previous
import jax
import jax.numpy as jnp
from jax.experimental import pallas as pl


def _gcn_kernel(x_ref, adj_ref, w1_ref, b1_ref, w2_ref, b2_ref, out_ref):
    adj = adj_ref[...]
    t = jnp.dot(x_ref[...], w1_ref[...],
                preferred_element_type=jnp.float32)
    h = jnp.dot(adj, t, preferred_element_type=jnp.float32) + b1_ref[...]
    g = jnp.where(h >= 0.0, h, 0.01 * h)
    n = adj.shape[0]
    r = jnp.sum(adj, axis=0, keepdims=True) * (1.0 / n)
    v = jnp.dot(r, g, preferred_element_type=jnp.float32)
    out_ref[0] = (jnp.dot(v, w2_ref[...],
                          preferred_element_type=jnp.float32)
                  + b2_ref[...])


def kernel(x, graph_batch, W1, b1, W2, b2):
    B, N, d_in = x.shape
    d_hid = W1.shape[1]
    d_out = W2.shape[1]
    x2 = x.reshape(B * N, d_in)
    adj2 = graph_batch.reshape(B * N, N)
    b1r = b1.reshape(1, d_hid)
    b2r = b2.reshape(1, d_out)
    return pl.pallas_call(
        _gcn_kernel,
        grid=(B,),
        in_specs=[
            pl.BlockSpec((N, d_in), lambda b: (b, 0)),
            pl.BlockSpec((N, N), lambda b: (b, 0)),
            pl.BlockSpec((d_in, d_hid), lambda b: (0, 0)),
            pl.BlockSpec((1, d_hid), lambda b: (0, 0)),
            pl.BlockSpec((d_hid, d_out), lambda b: (0, 0)),
            pl.BlockSpec((1, d_out), lambda b: (0, 0)),
        ],
        out_specs=pl.BlockSpec((1, 1, d_out), lambda b: (b, 0, 0)),
        out_shape=jax.ShapeDtypeStruct((B, 1, d_out), jnp.float32),
    )(x2, adj2, W1, b1r, W2, b2r).reshape(B, d_out)


# speedup vs baseline: 1.9259x; 1.0210x over previous
import jax
import jax.numpy as jnp
from jax.experimental import pallas as pl


def _gcn_kernel(x_ref, at_ref, ab_ref, w1_ref, b1_ref, w2_ref, b2_ref,
                out_ref):
    top = at_ref[0]
    bot = ab_ref[0]
    nh = top.shape[0]
    n = top.shape[1]
    t = jnp.dot(x_ref[0], w1_ref[...],
                preferred_element_type=jnp.float32)
    ht = jnp.dot(top, t, preferred_element_type=jnp.float32) + b1_ref[...]
    hb = jnp.dot(bot, t, preferred_element_type=jnp.float32) + b1_ref[...]
    gt = jnp.where(ht >= 0.0, ht, 0.01 * ht)
    gb = jnp.where(hb >= 0.0, hb, 0.01 * hb)
    r = (jnp.sum(top, axis=0, keepdims=True)
         + jnp.sum(bot, axis=0, keepdims=True)) * (1.0 / n)
    v = (jnp.dot(r[:, :nh], gt, preferred_element_type=jnp.float32)
         + jnp.dot(r[:, nh:], gb, preferred_element_type=jnp.float32))
    out_ref[0] = (jnp.dot(v, w2_ref[...],
                          preferred_element_type=jnp.float32)
                  + b2_ref[...])


def kernel(x, graph_batch, W1, b1, W2, b2):
    B, N, d_in = x.shape
    d_hid = W1.shape[1]
    d_out = W2.shape[1]
    NH = N // 2
    b1r = b1.reshape(1, d_hid)
    b2r = b2.reshape(1, d_out)
    return pl.pallas_call(
        _gcn_kernel,
        grid=(B,),
        in_specs=[
            pl.BlockSpec((1, N, d_in), lambda b: (b, 0, 0)),
            pl.BlockSpec((1, NH, N), lambda b: (b, 0, 0)),
            pl.BlockSpec((1, NH, N), lambda b: (b, 1, 0)),
            pl.BlockSpec((d_in, d_hid), lambda b: (0, 0)),
            pl.BlockSpec((1, d_hid), lambda b: (0, 0)),
            pl.BlockSpec((d_hid, d_out), lambda b: (0, 0)),
            pl.BlockSpec((1, d_out), lambda b: (0, 0)),
        ],
        out_specs=pl.BlockSpec((1, 1, d_out), lambda b: (b, 0, 0)),
        out_shape=jax.ShapeDtypeStruct((B, 1, d_out), jnp.float32),
    )(x, graph_batch, graph_batch, W1, b1r, W2, b2r).reshape(B, d_out)
